# baseline (speedup 1.0000x reference)
import functools

import jax
import jax.numpy as jnp
from jax.experimental import pallas as pl
from jax.experimental.pallas import tpu as pltpu


def _ce_body(x_hbm, out_ref, buf0, buf1, sems, *, GB, nj, K):
    def _copies(b, slot):
        src = x_hbm.at[pl.ds(b * GB, GB)]
        return (
            pltpu.make_async_copy(src.at[:, 0, :], buf0.at[slot], sems.at[slot, 0]),
            pltpu.make_async_copy(src.at[:, 1, :], buf1.at[slot], sems.at[slot, 1]),
        )

    for k in range(K):
        for c in _copies(k, k):
            c.start()

    def _step(b, acc):
        slot = jax.lax.rem(b, K)
        c0, c1 = _copies(b, slot)
        c0.wait()
        c1.wait()
        d = buf0[slot] - buf1[slot]
        sp = jnp.maximum(d, 0.0) + jnp.log1p(jnp.exp(-jnp.abs(d)))

        @pl.when(b + K < nj)
        def _refill():
            for c in _copies(b + K, slot):
                c.start()

        return acc + jnp.sum(sp)

    acc = jax.lax.fori_loop(0, nj, _step, jnp.float32(0.0))
    out_ref[...] = jnp.full((1, 1), acc, jnp.float32)


def kernel(logits):
    B, C = logits.shape
    G = B // 128
    dense = logits.reshape(G, 128, C).transpose(0, 2, 1)

    GB = 2048
    K = 8
    nj = G // GB

    partial = pl.pallas_call(
        functools.partial(_ce_body, GB=GB, nj=nj, K=K),
        out_shape=jax.ShapeDtypeStruct((1, 1), jnp.float32),
        in_specs=[pl.BlockSpec(memory_space=pltpu.MemorySpace.HBM)],
        out_specs=pl.BlockSpec(memory_space=pltpu.MemorySpace.VMEM),
        scratch_shapes=[
            pltpu.VMEM((K, GB, 128), jnp.float32),
            pltpu.VMEM((K, GB, 128), jnp.float32),
            pltpu.SemaphoreType.DMA((K, 2)),
        ],
    )(dense)
    return partial[0, 0] * (1.0 / B)


# speedup vs baseline: 1.0189x; 1.0189x over previous
import functools

import jax
import jax.numpy as jnp
from jax.experimental import pallas as pl
from jax.experimental.pallas import tpu as pltpu


def _ce_body(x_hbm, out_ref, buf0, buf1, sems, *, GB, nj, K):
    def _copies(b, slot):
        src = x_hbm.at[pl.ds(b * GB, GB)]
        return (
            pltpu.make_async_copy(src.at[:, 0, :], buf0.at[slot], sems.at[slot, 0]),
            pltpu.make_async_copy(src.at[:, 1, :], buf1.at[slot], sems.at[slot, 1]),
        )

    for k in range(K):
        for c in _copies(k, k):
            c.start()

    def _step(b, acc):
        slot = jax.lax.rem(b, K)
        c0, c1 = _copies(b, slot)
        c0.wait()
        c1.wait()
        d = buf0[slot] - buf1[slot]
        sp = jnp.maximum(d, 0.0) + jnp.log1p(jnp.exp(-jnp.abs(d)))

        @pl.when(b + K < nj)
        def _refill():
            for c in _copies(b + K, slot):
                c.start()

        return acc + jnp.sum(sp)

    acc = jax.lax.fori_loop(0, nj, _step, jnp.float32(0.0))
    out_ref[...] = jnp.full((1, 1), acc, jnp.float32)


def kernel(logits):
    B, C = logits.shape
    G = B // 128
    dense = logits.reshape(G, 128, C).transpose(0, 2, 1)

    GB = 2048
    K = 4
    nj = G // GB

    partial = pl.pallas_call(
        functools.partial(_ce_body, GB=GB, nj=nj, K=K),
        out_shape=jax.ShapeDtypeStruct((1, 1), jnp.float32),
        in_specs=[pl.BlockSpec(memory_space=pltpu.MemorySpace.HBM)],
        out_specs=pl.BlockSpec(memory_space=pltpu.MemorySpace.VMEM),
        scratch_shapes=[
            pltpu.VMEM((K, GB, 128), jnp.float32),
            pltpu.VMEM((K, GB, 128), jnp.float32),
            pltpu.SemaphoreType.DMA((K, 2)),
        ],
    )(dense)
    return partial[0, 0] * (1.0 / B)
